# baseline (speedup 1.0000x reference)
import functools

import numpy as np
import jax
import jax.numpy as jnp
from jax import lax
from jax.experimental import pallas as pl
from jax.experimental.pallas import tpu as pltpu
from jax.experimental.pallas import tpu_sc as plsc

_B = 1024
_D = 96
_S = 16
_NUM_T = 50
_CAN = 64
_NPIX = _CAN * _CAN
_NC = 2
_NS = 16
_NW = _NC * _NS
_SPW = _B // _NW

_SCALE = 1e-4
_LP_CONST = np.float32(_D * np.float32(-np.log(_SCALE) - 0.5 * np.log(2.0 * np.pi)))
_ENT_CONST = np.float32(_D * np.float32(0.5 * np.log(2.0 * np.pi * np.e * _SCALE ** 2)))
_MAGIC = np.float32(2.0 ** 23)
_INV49 = np.float32(1.0 / (_NUM_T - 1))


def _paint_body(x_hbm, sketch_hbm, lp_hbm, ent_hbm, sample_hbm,
                xv, canvas, outv, lpent_v, sk_sem, smp_sem):
    wid = lax.axis_index("s") * _NC + lax.axis_index("c")
    base = wid * _SPW

    lanes = lax.iota(jnp.int32, 16)
    wvec = jnp.full((16,), -0.07, jnp.float32)
    fill03 = jnp.full((16,), 0.3, jnp.float32)
    lp_fill = jnp.full((16,), _LP_CONST, jnp.float32)
    ent_fill = jnp.full((16,), _ENT_CONST, jnp.float32)

    lpent_v[pl.ds(0, 16)] = lp_fill
    lpent_v[pl.ds(16, 16)] = lp_fill
    lpent_v[pl.ds(32, 16)] = ent_fill
    lpent_v[pl.ds(48, 16)] = ent_fill
    pltpu.sync_copy(lpent_v.at[pl.ds(0, _SPW)], lp_hbm.at[pl.ds(base, _SPW)])
    pltpu.sync_copy(lpent_v.at[pl.ds(32, _SPW)], ent_hbm.at[pl.ds(base, _SPW)])

    pltpu.sync_copy(x_hbm.at[pl.ds(base * _D, _SPW * _D)], xv)

    def sig_body(k, carry):
        off = k * 16
        v = xv[pl.ds(off, 16)]
        e = jnp.exp(-jnp.abs(v))
        r = 1.0 / (1.0 + e)
        xv[pl.ds(off, 16)] = jnp.where(v >= 0.0, r, e / (1.0 + e))
        return carry

    lax.fori_loop(0, _SPW * _D // 16, sig_body, ())
    pltpu.async_copy(xv, sample_hbm.at[pl.ds(base * _D, _SPW * _D)], smp_sem)

    def init_body(k, carry):
        for u in range(8):
            canvas[pl.ds(k * 128 + u * 16, 16)] = fill03
        return carry

    lax.fori_loop(0, _NPIX // 128, init_body, ())

    def b_body(b, carry):
        boff = b * _D
        col_idx = lanes * 6 + boff
        p = [plsc.load_gather(xv, [col_idx + c]) * 64.0 for c in range(6)]
        p0x, p0y, p1x, p1y, p2x, p2y = p

        def t_body(i, tc):
            for u in range(2):
                tt = (i * 2 + u).astype(jnp.float32) * _INV49
                omt = 1.0 - tt
                c0 = omt * omt
                c1 = (2.0 * omt) * tt
                c2 = tt * tt
                bx = c0 * p0x + c1 * p1x + c2 * p2x
                by = c0 * p0y + c1 * p1y + c2 * p2y
                xi = ((bx + _MAGIC) - _MAGIC).astype(jnp.int32)
                yi = ((by + _MAGIC) - _MAGIC).astype(jnp.int32)
                xm = jnp.maximum(xi - 1, 0) * 64
                xz = jnp.minimum(xi, 63) * 64
                xp = jnp.minimum(xi + 1, 63) * 64
                ym = jnp.maximum(yi - 1, 0)
                yz = jnp.minimum(yi, 63)
                yp = jnp.minimum(yi + 1, 63)
                for xs in (xm, xz, xp):
                    for ys in (ym, yz, yp):
                        plsc.addupdate_scatter(canvas, [xs + ys], wvec)
            return tc

        lax.fori_loop(0, _NUM_T // 2, t_body, ())

        slot = lax.rem(b, 4)

        @pl.when(b >= 4)
        def _():
            pltpu.make_async_copy(
                outv.at[slot], sketch_hbm.at[base + b - 4], sk_sem.at[slot]
            ).wait()

        def d_body(k, dc):
            row = k * 2
            for u in range(8):
                off = k * 128 + u * 16
                v = jnp.maximum(canvas[pl.ds(off, 16)], 0.0)
                outv[slot, row + u // 4, pl.ds((u % 4) * 16, 16)] = v
                canvas[pl.ds(off, 16)] = fill03
            return dc

        lax.fori_loop(0, _NPIX // 128, d_body, ())
        pltpu.async_copy(outv.at[slot], sketch_hbm.at[base + b], sk_sem.at[slot])
        return carry

    lax.fori_loop(0, _SPW, b_body, ())

    for q in range(4):
        pltpu.make_async_copy(
            outv.at[q], sketch_hbm.at[base + _SPW - 4 + q], sk_sem.at[q]).wait()
    pltpu.make_async_copy(
        xv, sample_hbm.at[pl.ds(base * _D, _SPW * _D)], smp_sem).wait()


def _build_paint(interpret=False):
    return pl.kernel(
        _paint_body,
        out_type=(
            jax.ShapeDtypeStruct((_B, _CAN, _CAN), jnp.float32),
            jax.ShapeDtypeStruct((_B,), jnp.float32),
            jax.ShapeDtypeStruct((_B,), jnp.float32),
            jax.ShapeDtypeStruct((_B * _D,), jnp.float32),
        ),
        mesh=plsc.VectorSubcoreMesh(core_axis_name="c", subcore_axis_name="s"),
        scratch_types=[
            pltpu.VMEM((_SPW * _D,), jnp.float32),
            pltpu.VMEM((_NPIX,), jnp.float32),
            pltpu.VMEM((4, _CAN, _CAN), jnp.float32),
            pltpu.VMEM((64,), jnp.float32),
            pltpu.SemaphoreType.DMA((4,)),
            pltpu.SemaphoreType.DMA,
        ],
        compiler_params=pltpu.CompilerParams(
            needs_layout_passes=False, use_tc_tiling_on_sc=True),
        interpret=interpret,
    )


@functools.lru_cache(maxsize=None)
def _get_paint():
    return _build_paint()


def kernel(x, log_std):
    del log_std
    sk, lp, ent, sm = _get_paint()(x.reshape(-1))
    return (sk, lp, ent, sm.reshape(_B, _D))


# speedup vs baseline: 1.0251x; 1.0251x over previous
import functools

import numpy as np
import jax
import jax.numpy as jnp
from jax import lax
from jax.experimental import pallas as pl
from jax.experimental.pallas import tpu as pltpu
from jax.experimental.pallas import tpu_sc as plsc

_B = 1024
_D = 96
_S = 16
_NUM_T = 50
_CAN = 64
_NPIX = _CAN * _CAN
_NC = 2
_NS = 16
_NW = _NC * _NS
_SPW = _B // _NW

_SCALE = 1e-4
_LP_CONST = np.float32(_D * np.float32(-np.log(_SCALE) - 0.5 * np.log(2.0 * np.pi)))
_ENT_CONST = np.float32(_D * np.float32(0.5 * np.log(2.0 * np.pi * np.e * _SCALE ** 2)))
_MAGIC = np.float32(2.0 ** 23)
_INV49 = np.float32(1.0 / (_NUM_T - 1))


def _paint_body(x_hbm, sketch_hbm, lp_hbm, ent_hbm, sample_hbm,
                xv, canvas, outv, lpent_v, sk_sem, smp_sem):
    wid = lax.axis_index("s") * _NC + lax.axis_index("c")
    base = wid * _SPW

    lanes = lax.iota(jnp.int32, 16)
    wvec = jnp.full((16,), -0.07, jnp.float32)
    fill03 = jnp.full((16,), 0.3, jnp.float32)
    lp_fill = jnp.full((16,), _LP_CONST, jnp.float32)
    ent_fill = jnp.full((16,), _ENT_CONST, jnp.float32)

    lpent_v[pl.ds(0, 16)] = lp_fill
    lpent_v[pl.ds(16, 16)] = lp_fill
    lpent_v[pl.ds(32, 16)] = ent_fill
    lpent_v[pl.ds(48, 16)] = ent_fill
    pltpu.sync_copy(lpent_v.at[pl.ds(0, _SPW)], lp_hbm.at[pl.ds(base, _SPW)])
    pltpu.sync_copy(lpent_v.at[pl.ds(32, _SPW)], ent_hbm.at[pl.ds(base, _SPW)])

    pltpu.sync_copy(x_hbm.at[pl.ds(base * _D, _SPW * _D)], xv)

    def sig_body(k, carry):
        off = k * 16
        v = xv[pl.ds(off, 16)]
        e = jnp.exp(-jnp.abs(v))
        r = 1.0 / (1.0 + e)
        xv[pl.ds(off, 16)] = jnp.where(v >= 0.0, r, e / (1.0 + e))
        return carry

    lax.fori_loop(0, _SPW * _D // 16, sig_body, ())
    pltpu.async_copy(xv, sample_hbm.at[pl.ds(base * _D, _SPW * _D)], smp_sem)

    def init_body(k, carry):
        for u in range(8):
            canvas[pl.ds(k * 128 + u * 16, 16)] = fill03
        return carry

    lax.fori_loop(0, _NPIX // 128, init_body, ())

    def b_body(b, carry):
        boff = b * _D
        col_idx = lanes * 6 + boff
        p = [plsc.load_gather(xv, [col_idx + c]) * 64.0 for c in range(6)]
        p0x, p0y, p1x, p1y, p2x, p2y = p

        def t_body(i, tc):
            tt = i.astype(jnp.float32) * _INV49
            omt = 1.0 - tt
            c0 = omt * omt
            c1 = (2.0 * omt) * tt
            c2 = tt * tt
            bx = c0 * p0x + c1 * p1x + c2 * p2x
            by = c0 * p0y + c1 * p1y + c2 * p2y
            xi = ((bx + _MAGIC) - _MAGIC).astype(jnp.int32)
            yi = ((by + _MAGIC) - _MAGIC).astype(jnp.int32)
            xm = jnp.maximum(xi - 1, 0) * 64
            xz = jnp.minimum(xi, 63) * 64
            xp = jnp.minimum(xi + 1, 63) * 64
            ym = jnp.maximum(yi - 1, 0)
            yz = jnp.minimum(yi, 63)
            yp = jnp.minimum(yi + 1, 63)
            for xs in (xm, xz, xp):
                for ys in (ym, yz, yp):
                    plsc.addupdate_scatter(canvas, [xs + ys], wvec)
            return tc

        lax.fori_loop(0, _NUM_T, t_body, ())

        slot = lax.rem(b, 4)

        @pl.when(b >= 4)
        def _():
            pltpu.make_async_copy(
                outv.at[slot], sketch_hbm.at[base + b - 4], sk_sem.at[slot]
            ).wait()

        def d_body(k, dc):
            row = k * 2
            for u in range(8):
                off = k * 128 + u * 16
                v = jnp.maximum(canvas[pl.ds(off, 16)], 0.0)
                outv[slot, row + u // 4, pl.ds((u % 4) * 16, 16)] = v
                canvas[pl.ds(off, 16)] = fill03
            return dc

        lax.fori_loop(0, _NPIX // 128, d_body, ())
        pltpu.async_copy(outv.at[slot], sketch_hbm.at[base + b], sk_sem.at[slot])
        return carry

    lax.fori_loop(0, _SPW, b_body, ())

    for q in range(4):
        pltpu.make_async_copy(
            outv.at[q], sketch_hbm.at[base + _SPW - 4 + q], sk_sem.at[q]).wait()
    pltpu.make_async_copy(
        xv, sample_hbm.at[pl.ds(base * _D, _SPW * _D)], smp_sem).wait()


def _build_paint(interpret=False):
    return pl.kernel(
        _paint_body,
        out_type=(
            jax.ShapeDtypeStruct((_B, _CAN, _CAN), jnp.float32),
            jax.ShapeDtypeStruct((_B,), jnp.float32),
            jax.ShapeDtypeStruct((_B,), jnp.float32),
            jax.ShapeDtypeStruct((_B * _D,), jnp.float32),
        ),
        mesh=plsc.VectorSubcoreMesh(core_axis_name="c", subcore_axis_name="s"),
        scratch_types=[
            pltpu.VMEM((_SPW * _D,), jnp.float32),
            pltpu.VMEM((_NPIX,), jnp.float32),
            pltpu.VMEM((4, _CAN, _CAN), jnp.float32),
            pltpu.VMEM((64,), jnp.float32),
            pltpu.SemaphoreType.DMA((4,)),
            pltpu.SemaphoreType.DMA,
        ],
        compiler_params=pltpu.CompilerParams(
            needs_layout_passes=False, use_tc_tiling_on_sc=True),
        interpret=interpret,
    )


@functools.lru_cache(maxsize=None)
def _get_paint():
    return _build_paint()


def kernel(x, log_std):
    del log_std
    sk, lp, ent, sm = _get_paint()(x.reshape(-1))
    return (sk, lp, ent, sm.reshape(_B, _D))


# speedup vs baseline: 1.0941x; 1.0673x over previous
import functools

import numpy as np
import jax
import jax.numpy as jnp
from jax import lax
from jax.experimental import pallas as pl
from jax.experimental.pallas import tpu as pltpu
from jax.experimental.pallas import tpu_sc as plsc

_B = 1024
_D = 96
_S = 16
_NUM_T = 50
_CAN = 64
_NPIX = _CAN * _CAN
_NC = 2
_NS = 16
_NW = _NC * _NS
_SPW = _B // _NW

_SCALE = 1e-4
_LP_CONST = np.float32(_D * np.float32(-np.log(_SCALE) - 0.5 * np.log(2.0 * np.pi)))
_ENT_CONST = np.float32(_D * np.float32(0.5 * np.log(2.0 * np.pi * np.e * _SCALE ** 2)))
_MAGIC = np.float32(2.0 ** 23)
_INV49 = np.float32(1.0 / (_NUM_T - 1))


def _paint_body(x_hbm, sketch_hbm, lp_hbm, ent_hbm, sample_hbm,
                xv, canvas, outv, lpent_v, sk_sem, smp_sem):
    wid = lax.axis_index("s") * _NC + lax.axis_index("c")
    base = wid * _SPW

    lanes = lax.iota(jnp.int32, 16)
    wvec = jnp.full((16,), -0.07, jnp.float32)
    fill03 = jnp.full((16,), 0.3, jnp.float32)
    lp_fill = jnp.full((16,), _LP_CONST, jnp.float32)
    ent_fill = jnp.full((16,), _ENT_CONST, jnp.float32)

    lpent_v[pl.ds(0, 16)] = lp_fill
    lpent_v[pl.ds(16, 16)] = lp_fill
    lpent_v[pl.ds(32, 16)] = ent_fill
    lpent_v[pl.ds(48, 16)] = ent_fill
    pltpu.sync_copy(lpent_v.at[pl.ds(0, _SPW)], lp_hbm.at[pl.ds(base, _SPW)])
    pltpu.sync_copy(lpent_v.at[pl.ds(32, _SPW)], ent_hbm.at[pl.ds(base, _SPW)])

    pltpu.sync_copy(x_hbm.at[pl.ds(base * _D, _SPW * _D)], xv)

    def sig_body(k, carry):
        off = k * 16
        v = xv[pl.ds(off, 16)]
        e = jnp.exp(-jnp.abs(v))
        r = 1.0 / (1.0 + e)
        xv[pl.ds(off, 16)] = jnp.where(v >= 0.0, r, e / (1.0 + e))
        return carry

    lax.fori_loop(0, _SPW * _D // 16, sig_body, ())
    pltpu.async_copy(xv, sample_hbm.at[pl.ds(base * _D, _SPW * _D)], smp_sem)

    def init_body(k, carry):
        for u in range(8):
            canvas[pl.ds(k * 128 + u * 16, 16)] = fill03
        return carry

    lax.fori_loop(0, _NPIX // 128, init_body, ())

    def b_body(b, carry):
        boff = b * _D
        col_idx = lanes * 6 + boff
        p = [plsc.load_gather(xv, [col_idx + c]) * 64.0 for c in range(6)]
        p0x, p0y, p1x, p1y, p2x, p2y = p

        def t_body(i, tc):
            tt = i.astype(jnp.float32) * _INV49
            omt = 1.0 - tt
            c0 = omt * omt
            c1 = (2.0 * omt) * tt
            c2 = tt * tt
            bx = c0 * p0x + c1 * p1x + c2 * p2x
            by = c0 * p0y + c1 * p1y + c2 * p2y
            xi = ((bx + _MAGIC) - _MAGIC).astype(jnp.int32)
            yi = ((by + _MAGIC) - _MAGIC).astype(jnp.int32)
            xm = jnp.maximum(xi - 1, 0) * 64
            xz = jnp.minimum(xi, 63) * 64
            xp = jnp.minimum(xi + 1, 63) * 64
            ym = jnp.maximum(yi - 1, 0)
            yz = jnp.minimum(yi, 63)
            yp = jnp.minimum(yi + 1, 63)
            for xs in (xm, xz, xp):
                for ys in (ym, yz, yp):
                    plsc.addupdate_scatter(canvas, [xs + ys], wvec)
            return tc

        lax.fori_loop(0, _NUM_T, t_body, ())

        slot = lax.rem(b, 4)

        @pl.when(b >= 4)
        def _():
            pltpu.make_async_copy(
                outv.at[slot], sketch_hbm.at[base + b - 4], sk_sem.at[slot]
            ).wait()

        def d_body(k, dc):
            for u in range(8):
                off = k * 128 + u * 16
                v = jnp.maximum(canvas[pl.ds(off, 16)], 0.0)
                outv[slot, k, pl.ds(u * 16, 16)] = v
                canvas[pl.ds(off, 16)] = fill03
            return dc

        lax.fori_loop(0, _NPIX // 128, d_body, ())
        pltpu.async_copy(outv.at[slot], sketch_hbm.at[base + b], sk_sem.at[slot])
        return carry

    lax.fori_loop(0, _SPW, b_body, ())

    for q in range(4):
        pltpu.make_async_copy(
            outv.at[q], sketch_hbm.at[base + _SPW - 4 + q], sk_sem.at[q]).wait()
    pltpu.make_async_copy(
        xv, sample_hbm.at[pl.ds(base * _D, _SPW * _D)], smp_sem).wait()


def _build_paint(interpret=False):
    return pl.kernel(
        _paint_body,
        out_type=(
            jax.ShapeDtypeStruct((_B, 32, 128), jnp.float32),
            jax.ShapeDtypeStruct((_B,), jnp.float32),
            jax.ShapeDtypeStruct((_B,), jnp.float32),
            jax.ShapeDtypeStruct((_B * _D,), jnp.float32),
        ),
        mesh=plsc.VectorSubcoreMesh(core_axis_name="c", subcore_axis_name="s"),
        scratch_types=[
            pltpu.VMEM((_SPW * _D,), jnp.float32),
            pltpu.VMEM((_NPIX,), jnp.float32),
            pltpu.VMEM((4, 32, 128), jnp.float32),
            pltpu.VMEM((64,), jnp.float32),
            pltpu.SemaphoreType.DMA((4,)),
            pltpu.SemaphoreType.DMA,
        ],
        compiler_params=pltpu.CompilerParams(
            needs_layout_passes=False, use_tc_tiling_on_sc=True),
        interpret=interpret,
    )


@functools.lru_cache(maxsize=None)
def _get_paint():
    return _build_paint()


def kernel(x, log_std):
    del log_std
    sk, lp, ent, sm = _get_paint()(x.reshape(-1))
    return (sk.reshape(_B, _CAN, _CAN), lp, ent, sm.reshape(_B, _D))


# speedup vs baseline: 1.0946x; 1.0005x over previous
import functools

import numpy as np
import jax
import jax.numpy as jnp
from jax import lax
from jax.experimental import pallas as pl
from jax.experimental.pallas import tpu as pltpu
from jax.experimental.pallas import tpu_sc as plsc

_B = 1024
_D = 96
_S = 16
_NUM_T = 50
_CAN = 64
_NPIX = _CAN * _CAN
_NC = 2
_NS = 16
_NW = _NC * _NS
_SPW = _B // _NW

_SCALE = 1e-4
_LP_CONST = np.float32(_D * np.float32(-np.log(_SCALE) - 0.5 * np.log(2.0 * np.pi)))
_ENT_CONST = np.float32(_D * np.float32(0.5 * np.log(2.0 * np.pi * np.e * _SCALE ** 2)))
_MAGIC = np.float32(2.0 ** 23)
_INV49 = np.float32(1.0 / (_NUM_T - 1))


def _paint_body(x_hbm, sketch_hbm, lp_hbm, ent_hbm, sample_hbm,
                xv, canvas, outv, lpent_v, sk_sem, smp_sem):
    wid = lax.axis_index("s") * _NC + lax.axis_index("c")
    base = wid * _SPW

    lanes = lax.iota(jnp.int32, 16)
    wvec = jnp.full((16,), -0.07, jnp.float32)
    fill03 = jnp.full((16,), 0.3, jnp.float32)
    lp_fill = jnp.full((16,), _LP_CONST, jnp.float32)
    ent_fill = jnp.full((16,), _ENT_CONST, jnp.float32)

    lpent_v[pl.ds(0, 16)] = lp_fill
    lpent_v[pl.ds(16, 16)] = lp_fill
    lpent_v[pl.ds(32, 16)] = ent_fill
    lpent_v[pl.ds(48, 16)] = ent_fill
    pltpu.sync_copy(lpent_v.at[pl.ds(0, _SPW)], lp_hbm.at[pl.ds(base, _SPW)])
    pltpu.sync_copy(lpent_v.at[pl.ds(32, _SPW)], ent_hbm.at[pl.ds(base, _SPW)])

    pltpu.sync_copy(x_hbm.at[pl.ds(base * _D, _SPW * _D)], xv)

    def sig_body(k, carry):
        off = k * 16
        v = xv[pl.ds(off, 16)]
        e = jnp.exp(-jnp.abs(v))
        r = 1.0 / (1.0 + e)
        xv[pl.ds(off, 16)] = jnp.where(v >= 0.0, r, e / (1.0 + e))
        return carry

    lax.fori_loop(0, _SPW * _D // 16, sig_body, ())
    pltpu.async_copy(xv, sample_hbm.at[pl.ds(base * _D, _SPW * _D)], smp_sem)

    def init_body(k, carry):
        for u in range(8):
            canvas[pl.ds(k * 128 + u * 16, 16)] = fill03
        return carry

    lax.fori_loop(0, _NPIX // 128, init_body, ())

    def b_body(b, carry):
        boff = b * _D
        col_idx = lanes * 6 + boff
        p = [plsc.load_gather(xv, [col_idx + c]) * 64.0 for c in range(6)]
        p0x, p0y, p1x, p1y, p2x, p2y = p

        def t_body(i, tc):
            tt = i.astype(jnp.float32) * _INV49
            omt = 1.0 - tt
            c0 = omt * omt
            c1 = (2.0 * omt) * tt
            c2 = tt * tt
            bx = c0 * p0x + c1 * p1x + c2 * p2x
            by = c0 * p0y + c1 * p1y + c2 * p2y
            xi = ((bx + _MAGIC) - _MAGIC).astype(jnp.int32)
            yi = ((by + _MAGIC) - _MAGIC).astype(jnp.int32)
            xm = jnp.maximum(xi - 1, 0) * 64
            xz = jnp.minimum(xi, 63) * 64
            xp = jnp.minimum(xi + 1, 63) * 64
            ym = jnp.maximum(yi - 1, 0)
            yz = jnp.minimum(yi, 63)
            yp = jnp.minimum(yi + 1, 63)
            for xs in (xm, xz, xp):
                for ys in (ym, yz, yp):
                    plsc.addupdate_scatter(canvas, [xs + ys], wvec)
            return tc

        lax.fori_loop(0, _NUM_T, t_body, ())

        slot = lax.rem(b, 4)

        @pl.when(b >= 4)
        def _():
            pltpu.make_async_copy(
                outv.at[slot], sketch_hbm.at[base + b - 4], sk_sem.at[slot]
            ).wait()

        def d_body(k, dc):
            for u in range(8):
                off = k * 128 + u * 16
                v = jnp.maximum(canvas[pl.ds(off, 16)], 0.0)
                outv[slot, k, pl.ds(u * 16, 16)] = v
                canvas[pl.ds(off, 16)] = fill03
            return dc

        lax.fori_loop(0, _NPIX // 128, d_body, ())
        pltpu.async_copy(outv.at[slot], sketch_hbm.at[base + b], sk_sem.at[slot])
        return carry

    lax.fori_loop(0, _SPW, b_body, ())

    for q in range(4):
        pltpu.make_async_copy(
            outv.at[q], sketch_hbm.at[base + _SPW - 4 + q], sk_sem.at[q]).wait()
    pltpu.make_async_copy(
        xv, sample_hbm.at[pl.ds(base * _D, _SPW * _D)], smp_sem).wait()


def _build_paint(interpret=False):
    return pl.kernel(
        _paint_body,
        out_type=(
            jax.ShapeDtypeStruct((_B, 32, 128), jnp.float32),
            jax.ShapeDtypeStruct((_B,), jnp.float32),
            jax.ShapeDtypeStruct((_B,), jnp.float32),
            jax.ShapeDtypeStruct((_B * _D,), jnp.float32),
        ),
        mesh=plsc.VectorSubcoreMesh(core_axis_name="c", subcore_axis_name="s"),
        scratch_types=[
            pltpu.VMEM((_SPW * _D,), jnp.float32),
            pltpu.VMEM((_NPIX,), jnp.float32),
            pltpu.VMEM((4, 32, 128), jnp.float32),
            pltpu.VMEM((64,), jnp.float32),
            pltpu.SemaphoreType.DMA((4,)),
            pltpu.SemaphoreType.DMA,
        ],
        compiler_params=pltpu.CompilerParams(needs_layout_passes=False),
        interpret=interpret,
    )


@functools.lru_cache(maxsize=None)
def _get_paint():
    return _build_paint()


def kernel(x, log_std):
    del log_std
    sk, lp, ent, sm = _get_paint()(x.reshape(-1))
    return (sk.reshape(_B, _CAN, _CAN), lp, ent, sm.reshape(_B, _D))


# speedup vs baseline: 1.3285x; 1.2137x over previous
import functools

import numpy as np
import jax
import jax.numpy as jnp
from jax import lax
from jax.experimental import pallas as pl
from jax.experimental.pallas import tpu as pltpu
from jax.experimental.pallas import tpu_sc as plsc

_B = 1024
_D = 96
_S = 16
_NUM_T = 50
_CAN = 64
_NPIX = _CAN * _CAN
_NC = 2
_NS = 16
_NW = _NC * _NS
_SPW = _B // _NW

_SCALE = 1e-4
_LP_CONST = np.float32(_D * np.float32(-np.log(_SCALE) - 0.5 * np.log(2.0 * np.pi)))
_ENT_CONST = np.float32(_D * np.float32(0.5 * np.log(2.0 * np.pi * np.e * _SCALE ** 2)))
_MAGIC = np.float32(2.0 ** 23)
_INV49 = np.float32(1.0 / (_NUM_T - 1))


def _paint_body(x_hbm, sketch_hbm, lp_hbm, ent_hbm, sample_hbm,
                xv, canvas, outv, lpent_v, smp_sem):
    wid = lax.axis_index("s") * _NC + lax.axis_index("c")
    base = wid * _SPW

    lanes = lax.iota(jnp.int32, 16)
    wvec = jnp.full((16,), -0.07, jnp.float32)
    fill03 = jnp.full((16,), 0.3, jnp.float32)
    lp_fill = jnp.full((16,), _LP_CONST, jnp.float32)
    ent_fill = jnp.full((16,), _ENT_CONST, jnp.float32)

    lpent_v[pl.ds(0, 16)] = lp_fill
    lpent_v[pl.ds(16, 16)] = lp_fill
    lpent_v[pl.ds(32, 16)] = ent_fill
    lpent_v[pl.ds(48, 16)] = ent_fill
    pltpu.sync_copy(lpent_v.at[pl.ds(0, _SPW)], lp_hbm.at[pl.ds(base, _SPW)])
    pltpu.sync_copy(lpent_v.at[pl.ds(32, _SPW)], ent_hbm.at[pl.ds(base, _SPW)])

    pltpu.sync_copy(x_hbm.at[pl.ds(base * _D, _SPW * _D)], xv)

    def sig_body(k, carry):
        off = k * 16
        v = xv[pl.ds(off, 16)]
        e = jnp.exp(-jnp.abs(v))
        r = 1.0 / (1.0 + e)
        xv[pl.ds(off, 16)] = jnp.where(v >= 0.0, r, e / (1.0 + e))
        return carry

    lax.fori_loop(0, _SPW * _D // 16, sig_body, ())
    pltpu.async_copy(xv, sample_hbm.at[pl.ds(base * _D, _SPW * _D)], smp_sem)

    def init_body(k, carry):
        for u in range(8):
            canvas[pl.ds(k * 128 + u * 16, 16)] = fill03
        return carry

    lax.fori_loop(0, _NPIX // 128, init_body, ())

    def b_body(b, carry):
        boff = b * _D
        col_idx = lanes * 6 + boff
        p = [plsc.load_gather(xv, [col_idx + c]) * 64.0 for c in range(6)]
        p0x, p0y, p1x, p1y, p2x, p2y = p

        def t_body(i, tc):
            tt = i.astype(jnp.float32) * _INV49
            omt = 1.0 - tt
            c0 = omt * omt
            c1 = (2.0 * omt) * tt
            c2 = tt * tt
            bx = c0 * p0x + c1 * p1x + c2 * p2x
            by = c0 * p0y + c1 * p1y + c2 * p2y
            xi = ((bx + _MAGIC) - _MAGIC).astype(jnp.int32)
            yi = ((by + _MAGIC) - _MAGIC).astype(jnp.int32)
            xm = jnp.maximum(xi - 1, 0) * 64
            xz = jnp.minimum(xi, 63) * 64
            xp = jnp.minimum(xi + 1, 63) * 64
            ym = jnp.maximum(yi - 1, 0)
            yz = jnp.minimum(yi, 63)
            yp = jnp.minimum(yi + 1, 63)
            for xs in (xm, xz, xp):
                for ys in (ym, yz, yp):
                    plsc.addupdate_scatter(canvas, [xs + ys], wvec)
            return tc

        lax.fori_loop(0, _NUM_T, t_body, ())

        def d_body(k, dc):
            for u in range(8):
                off = k * 128 + u * 16
                v = jnp.maximum(canvas[pl.ds(off, 16)], 0.0)
                outv[k, pl.ds(u * 16, 16)] = v
                canvas[pl.ds(off, 16)] = fill03
            return dc

        lax.fori_loop(0, _NPIX // 128, d_body, ())
        pltpu.sync_copy(outv, sketch_hbm.at[base + b])
        return carry

    lax.fori_loop(0, _SPW, b_body, ())

    pltpu.make_async_copy(
        xv, sample_hbm.at[pl.ds(base * _D, _SPW * _D)], smp_sem).wait()


def _build_paint(interpret=False):
    return pl.kernel(
        _paint_body,
        out_type=(
            jax.ShapeDtypeStruct((_B, 32, 128), jnp.float32),
            jax.ShapeDtypeStruct((_B,), jnp.float32),
            jax.ShapeDtypeStruct((_B,), jnp.float32),
            jax.ShapeDtypeStruct((_B * _D,), jnp.float32),
        ),
        mesh=plsc.VectorSubcoreMesh(core_axis_name="c", subcore_axis_name="s"),
        scratch_types=[
            pltpu.VMEM((_SPW * _D,), jnp.float32),
            pltpu.VMEM((_NPIX,), jnp.float32),
            pltpu.VMEM((32, 128), jnp.float32),
            pltpu.VMEM((64,), jnp.float32),
            pltpu.SemaphoreType.DMA,
        ],
        compiler_params=pltpu.CompilerParams(needs_layout_passes=False),
        interpret=interpret,
    )


@functools.lru_cache(maxsize=None)
def _get_paint():
    return _build_paint()


def kernel(x, log_std):
    del log_std
    sk, lp, ent, sm = _get_paint()(x.reshape(-1))
    return (sk.reshape(_B, _CAN, _CAN), lp, ent, sm.reshape(_B, _D))
